# baseline (speedup 1.0000x reference)
import jax
import jax.numpy as jnp
from jax.experimental import pallas as pl


_BLOCK_S = 512


def _add_kernel(x_ref, pos_ref, out_ref):
    out_ref[...] = x_ref[...] + pos_ref[...]


def kernel(x, pos_emb):
    batch, seq_len, d_model = x.shape
    pos = pos_emb[:seq_len]
    grid = (batch, seq_len // _BLOCK_S)
    return pl.pallas_call(
        _add_kernel,
        grid=grid,
        in_specs=[
            pl.BlockSpec((1, _BLOCK_S, d_model), lambda b, s: (b, s, 0)),
            pl.BlockSpec((_BLOCK_S, d_model), lambda b, s: (s, 0)),
        ],
        out_specs=pl.BlockSpec((1, _BLOCK_S, d_model), lambda b, s: (b, s, 0)),
        out_shape=jax.ShapeDtypeStruct(x.shape, x.dtype),
    )(x, pos)


# speedup vs baseline: 1.3521x; 1.3521x over previous
import jax
import jax.numpy as jnp
from jax.experimental import pallas as pl


_BLOCK_S = 512


def _add_kernel(x_ref, pos_ref, out_ref):
    out_ref[...] = x_ref[...] + pos_ref[...]


def kernel(x, pos_emb):
    batch, seq_len, d_model = x.shape
    pos = pos_emb[:seq_len]
    grid = (seq_len // _BLOCK_S,)
    return pl.pallas_call(
        _add_kernel,
        grid=grid,
        in_specs=[
            pl.BlockSpec((batch, _BLOCK_S, d_model), lambda s: (0, s, 0)),
            pl.BlockSpec((_BLOCK_S, d_model), lambda s: (s, 0)),
        ],
        out_specs=pl.BlockSpec((batch, _BLOCK_S, d_model), lambda s: (0, s, 0)),
        out_shape=jax.ShapeDtypeStruct(x.shape, x.dtype),
    )(x, pos)
